# Initial kernel scaffold; baseline (speedup 1.0000x reference)
import jax
import jax.numpy as jnp
from jax.experimental import pallas as pl


def kernel(sparse_input, table):
    raise NotImplementedError("write your pallas kernel here")


# speedup vs baseline: 2.4906x; 2.4906x over previous
import functools

import jax
import jax.numpy as jnp
from jax import lax
from jax.experimental import pallas as pl
from jax.experimental.pallas import tpu as pltpu
from jax.experimental.pallas import tpu_sc as plsc

_NC = 2
_NS = 16
_NW = _NC * _NS
_CHUNK = 2048


def _gather_body(idx_hbm, tab_hbm, out_hbm, idx_v, rows_v, sem, *, per_w, n_chunks):
    wid = lax.axis_index("s") * _NC + lax.axis_index("c")
    base = wid * per_w

    def body(i, carry):
        off = base + i * _CHUNK
        pltpu.sync_copy(idx_hbm.at[pl.ds(off, _CHUNK)], idx_v)
        pltpu.async_copy(tab_hbm.at[idx_v], rows_v, sem).wait()
        pltpu.sync_copy(rows_v, out_hbm.at[pl.ds(off, _CHUNK)])
        return carry

    lax.fori_loop(0, n_chunks, body, 0)


def kernel(sparse_input, table):
    B, N = sparse_input.shape
    V, D = table.shape
    total = B * N
    assert total % (_NW * _CHUNK) == 0
    per_w = total // _NW
    n_chunks = per_w // _CHUNK

    flat_idx = sparse_input.reshape(total)
    mesh = plsc.VectorSubcoreMesh(core_axis_name="c", subcore_axis_name="s")

    run = functools.partial(
        pl.kernel,
        out_type=jax.ShapeDtypeStruct((total, D), jnp.float32),
        mesh=mesh,
        scratch_types=[
            pltpu.VMEM((_CHUNK,), jnp.int32),
            pltpu.VMEM((_CHUNK, D), jnp.float32),
            pltpu.SemaphoreType.DMA,
        ],
        compiler_params=pltpu.CompilerParams(use_tc_tiling_on_sc=False),
    )(functools.partial(_gather_body, per_w=per_w, n_chunks=n_chunks))

    out = run(flat_idx, table)
    return out.reshape(B, N, D)


# speedup vs baseline: 2.5624x; 1.0288x over previous
import functools

import jax
import jax.numpy as jnp
from jax import lax
from jax.experimental import pallas as pl
from jax.experimental.pallas import tpu as pltpu
from jax.experimental.pallas import tpu_sc as plsc

_NC = 2
_NS = 16
_NW = _NC * _NS
_CHUNK = 2048


def _gather_body(idx_hbm, tab_hbm, out_hbm,
                 idx_v0, idx_v1, rows_v0, rows_v1,
                 isem0, isem1, gsem0, gsem1, osem0, osem1,
                 *, per_w, n_chunks):
    wid = lax.axis_index("s") * _NC + lax.axis_index("c")
    base = wid * per_w
    n_outer = n_chunks // 2

    idx_v = (idx_v0, idx_v1)
    rows_v = (rows_v0, rows_v1)
    isem = (isem0, isem1)
    gsem = (gsem0, gsem1)
    osem = (osem0, osem1)

    def idx_slice(j):
        return idx_hbm.at[pl.ds(base + j * _CHUNK, _CHUNK)]

    def out_slice(j):
        return out_hbm.at[pl.ds(base + j * _CHUNK, _CHUNK)]

    pltpu.async_copy(idx_slice(0), idx_v[0], isem[0])
    pltpu.async_copy(idx_slice(1), idx_v[1], isem[1])

    def body(g, carry):
        j0 = g * 2
        for s in range(2):
            j = j0 + s
            pltpu.make_async_copy(idx_slice(j), idx_v[s], isem[s]).wait()

            @pl.when(g != 0)
            def _():
                pltpu.make_async_copy(rows_v[s], out_slice(j - 2), osem[s]).wait()

            pltpu.async_copy(tab_hbm.at[idx_v[s]], rows_v[s], gsem[s])
        for s in range(2):
            j = j0 + s
            pltpu.make_async_copy(tab_hbm.at[idx_v[s]], rows_v[s], gsem[s]).wait()
            pltpu.async_copy(rows_v[s], out_slice(j), osem[s])

            @pl.when(g != n_outer - 1)
            def _():
                pltpu.async_copy(idx_slice(j + 2), idx_v[s], isem[s])

        return carry

    lax.fori_loop(0, n_outer, body, 0)

    last = n_chunks - 2
    pltpu.make_async_copy(rows_v[0], out_slice(last), osem[0]).wait()
    pltpu.make_async_copy(rows_v[1], out_slice(last + 1), osem[1]).wait()


def kernel(sparse_input, table):
    B, N = sparse_input.shape
    V, D = table.shape
    total = B * N
    assert total % (_NW * _CHUNK * 2) == 0
    per_w = total // _NW
    n_chunks = per_w // _CHUNK

    flat_idx = sparse_input.reshape(total)
    mesh = plsc.VectorSubcoreMesh(core_axis_name="c", subcore_axis_name="s")

    run = functools.partial(
        pl.kernel,
        out_type=jax.ShapeDtypeStruct((total, D), jnp.float32),
        mesh=mesh,
        scratch_types=[
            pltpu.VMEM((_CHUNK,), jnp.int32),
            pltpu.VMEM((_CHUNK,), jnp.int32),
            pltpu.VMEM((_CHUNK, D), jnp.float32),
            pltpu.VMEM((_CHUNK, D), jnp.float32),
            pltpu.SemaphoreType.DMA,
            pltpu.SemaphoreType.DMA,
            pltpu.SemaphoreType.DMA,
            pltpu.SemaphoreType.DMA,
            pltpu.SemaphoreType.DMA,
            pltpu.SemaphoreType.DMA,
        ],
        compiler_params=pltpu.CompilerParams(use_tc_tiling_on_sc=False),
    )(functools.partial(_gather_body, per_w=per_w, n_chunks=n_chunks))

    out = run(flat_idx, table)
    return out.reshape(B, N, D)


# speedup vs baseline: 2.5633x; 1.0004x over previous
import functools

import jax
import jax.numpy as jnp
from jax import lax
from jax.experimental import pallas as pl
from jax.experimental.pallas import tpu as pltpu
from jax.experimental.pallas import tpu_sc as plsc

_NC = 2
_NS = 16
_NW = _NC * _NS
_CHUNK = 2048


def _gather_body(idx_hbm, tab_hbm, out_hbm,
                 idx_v0, idx_v1, rows_v0, rows_v1,
                 isem0, isem1, gsem0, gsem1, osem0, osem1,
                 *, per_w, n_chunks):
    wid = lax.axis_index("s") * _NC + lax.axis_index("c")
    base = wid * per_w
    n_outer = n_chunks // 2

    idx_v = (idx_v0, idx_v1)
    rows_v = (rows_v0, rows_v1)
    isem = (isem0, isem1)
    gsem = (gsem0, gsem1)
    osem = (osem0, osem1)

    def idx_slice(j):
        return idx_hbm.at[pl.ds(base + j * _CHUNK, _CHUNK)]

    def out_slice(j):
        return out_hbm.at[pl.ds(base + j * _CHUNK, _CHUNK)]

    pltpu.async_copy(idx_slice(0), idx_v[0], isem[0])
    pltpu.async_copy(idx_slice(1), idx_v[1], isem[1])

    def body(g, carry):
        j0 = g * 2
        for s in range(2):
            j = j0 + s
            pltpu.make_async_copy(idx_slice(j), idx_v[s], isem[s]).wait()

            @pl.when(g != 0)
            def _():
                pltpu.make_async_copy(rows_v[s], out_slice(j - 2), osem[s]).wait()

            pltpu.async_copy(tab_hbm.at[idx_v[s]], rows_v[s], gsem[s])
        for s in range(2):
            j = j0 + s
            pltpu.make_async_copy(tab_hbm.at[idx_v[s]], rows_v[s], gsem[s]).wait()
            pltpu.async_copy(rows_v[s], out_slice(j), osem[s])

            @pl.when(g != n_outer - 1)
            def _():
                pltpu.async_copy(idx_slice(j + 2), idx_v[s], isem[s])

        return carry

    lax.fori_loop(0, n_outer, body, 0)

    last = n_chunks - 2
    pltpu.make_async_copy(rows_v[0], out_slice(last), osem[0]).wait()
    pltpu.make_async_copy(rows_v[1], out_slice(last + 1), osem[1]).wait()


def kernel(sparse_input, table):
    B, N = sparse_input.shape
    V, D = table.shape
    total = B * N
    assert total % (_NW * _CHUNK * 2) == 0
    per_w = total // _NW
    n_chunks = per_w // _CHUNK

    flat_idx = sparse_input.reshape(total)
    mesh = plsc.VectorSubcoreMesh(core_axis_name="c", subcore_axis_name="s")

    run = functools.partial(
        pl.kernel,
        out_type=jax.ShapeDtypeStruct((total, D), jnp.float32),
        mesh=mesh,
        scratch_types=[
            pltpu.VMEM((_CHUNK,), jnp.int32),
            pltpu.VMEM((_CHUNK,), jnp.int32),
            pltpu.VMEM((_CHUNK, D), jnp.float32),
            pltpu.VMEM((_CHUNK, D), jnp.float32),
            pltpu.SemaphoreType.DMA,
            pltpu.SemaphoreType.DMA,
            pltpu.SemaphoreType.DMA,
            pltpu.SemaphoreType.DMA,
            pltpu.SemaphoreType.DMA,
            pltpu.SemaphoreType.DMA,
        ],
        compiler_params=pltpu.CompilerParams(use_tc_tiling_on_sc=False, disable_bounds_checks=True),
    )(functools.partial(_gather_body, per_w=per_w, n_chunks=n_chunks))

    out = run(flat_idx, table)
    return out.reshape(B, N, D)


# speedup vs baseline: 2.5677x; 1.0017x over previous
import functools

import jax
import jax.numpy as jnp
from jax import lax
from jax.experimental import pallas as pl
from jax.experimental.pallas import tpu as pltpu
from jax.experimental.pallas import tpu_sc as plsc

_NC = 2
_NS = 16
_NW = _NC * _NS
_CHUNK = 1280
_NSLOTS = 4


def _gather_body(idx_hbm, tab_hbm, out_hbm, *refs, per_w, n_chunks):
    wid = lax.axis_index("s") * _NC + lax.axis_index("c")
    base = wid * per_w
    n_outer = n_chunks // _NSLOTS

    idx_v = refs[0:_NSLOTS]
    rows_v = refs[_NSLOTS:2 * _NSLOTS]
    isem = refs[2 * _NSLOTS:3 * _NSLOTS]
    gsem = refs[3 * _NSLOTS:4 * _NSLOTS]
    osem = refs[4 * _NSLOTS:5 * _NSLOTS]

    def idx_slice(j):
        return idx_hbm.at[pl.ds(base + j * _CHUNK, _CHUNK)]

    def out_slice(j):
        return out_hbm.at[pl.ds(base + j * _CHUNK, _CHUNK)]

    for s in range(_NSLOTS):
        pltpu.async_copy(idx_slice(s), idx_v[s], isem[s])

    def body(g, carry):
        j0 = g * _NSLOTS
        for s in range(_NSLOTS):
            j = j0 + s
            pltpu.make_async_copy(idx_slice(j), idx_v[s], isem[s]).wait()

            @pl.when(g != 0)
            def _():
                pltpu.make_async_copy(
                    rows_v[s], out_slice(j - _NSLOTS), osem[s]).wait()

            pltpu.async_copy(tab_hbm.at[idx_v[s]], rows_v[s], gsem[s])
        for s in range(_NSLOTS):
            j = j0 + s
            pltpu.make_async_copy(tab_hbm.at[idx_v[s]], rows_v[s], gsem[s]).wait()
            pltpu.async_copy(rows_v[s], out_slice(j), osem[s])

            @pl.when(g != n_outer - 1)
            def _():
                pltpu.async_copy(idx_slice(j + _NSLOTS), idx_v[s], isem[s])

        return carry

    lax.fori_loop(0, n_outer, body, 0)

    last = n_chunks - _NSLOTS
    for s in range(_NSLOTS):
        pltpu.make_async_copy(rows_v[s], out_slice(last + s), osem[s]).wait()


def kernel(sparse_input, table):
    B, N = sparse_input.shape
    V, D = table.shape
    total = B * N
    assert total % (_NW * _CHUNK * _NSLOTS) == 0
    per_w = total // _NW
    n_chunks = per_w // _CHUNK

    flat_idx = sparse_input.reshape(total)
    mesh = plsc.VectorSubcoreMesh(core_axis_name="c", subcore_axis_name="s")

    run = functools.partial(
        pl.kernel,
        out_type=jax.ShapeDtypeStruct((total, D), jnp.float32),
        mesh=mesh,
        scratch_types=(
            [pltpu.VMEM((_CHUNK,), jnp.int32)] * _NSLOTS
            + [pltpu.VMEM((_CHUNK, D), jnp.float32)] * _NSLOTS
            + [pltpu.SemaphoreType.DMA] * (3 * _NSLOTS)
        ),
        compiler_params=pltpu.CompilerParams(
            use_tc_tiling_on_sc=False, disable_bounds_checks=True),
    )(functools.partial(_gather_body, per_w=per_w, n_chunks=n_chunks))

    out = run(flat_idx, table)
    return out.reshape(B, N, D)
